# baseline (speedup 1.0000x reference)
import functools

import jax
import jax.numpy as jnp
from jax import lax
from jax.experimental import pallas as pl
from jax.experimental.pallas import tpu as pltpu
from jax.experimental.pallas import tpu_sc as plsc

D_MODEL = 128
SEQ = 50
LANES = 16
NUM_WORKERS = 32
CHUNK = 128
NBUF = 5


def _positional_encoding(seq, d_model):
    pos = jnp.arange(seq, dtype=jnp.float32)[:, None]
    i = jnp.arange(0, d_model, 2, dtype=jnp.float32)
    div = jnp.exp(-i * (jnp.log(10000.0) / d_model))
    ang = pos * div
    pe = jnp.zeros((seq, d_model), dtype=jnp.float32)
    pe = pe.at[:, 0::2].set(jnp.sin(ang))
    pe = pe.at[:, 1::2].set(jnp.cos(ang))
    return pe


def _make_sc_kernel(batch, chunks_per_plane, chunks_per_w):
    mesh = plsc.VectorSubcoreMesh(core_axis_name="c", subcore_axis_name="s")
    n_dreg = D_MODEL // LANES
    assert chunks_per_w % NBUF == 0

    @functools.partial(
        pl.kernel,
        mesh=mesh,
        out_type=jax.ShapeDtypeStruct((SEQ, batch, D_MODEL), jnp.float32),
        scratch_types=[
            pltpu.VMEM((SEQ, D_MODEL), jnp.float32),
        ]
        + [pltpu.VMEM((CHUNK,), jnp.int32) for _ in range(NBUF)]
        + [pltpu.VMEM((CHUNK, D_MODEL), jnp.float32) for _ in range(NBUF)]
        + [pltpu.SemaphoreType.DMA for _ in range(3 * NBUF)],
    )
    def sc_embed(x_hbm, tab_hbm, pe_hbm, out_hbm, pe_v, *bufs_sems):
        ibufs = bufs_sems[:NBUF]
        bufs = bufs_sems[NBUF:2 * NBUF]
        isem = bufs_sems[2 * NBUF:3 * NBUF]
        gsem = bufs_sems[3 * NBUF:4 * NBUF]
        ssem = bufs_sems[4 * NBUF:]
        cid = lax.axis_index("c")
        sid = lax.axis_index("s")
        w = sid * 2 + cid
        pltpu.sync_copy(pe_hbm, pe_v)
        gc0 = w * chunks_per_w

        def start_idx(b, c):
            pltpu.async_copy(x_hbm.at[w, c], ibufs[b], isem[b])

        def wait_idx(b):
            pltpu.make_async_copy(x_hbm.at[w, 0], ibufs[b], isem[b]).wait()

        def start_gather(b):
            pltpu.async_copy(tab_hbm.at[ibufs[b]], bufs[b], gsem[b])

        def wait_gather(b):
            pltpu.make_async_copy(tab_hbm.at[ibufs[b]], bufs[b],
                                  gsem[b]).wait()

        def start_scatter(b, c):
            gc = gc0 + c
            s = lax.div(gc, chunks_per_plane)
            j = lax.rem(gc, chunks_per_plane)
            pltpu.async_copy(bufs[b], out_hbm.at[s, pl.ds(j * CHUNK, CHUNK)],
                             ssem[b])

        def wait_scatter(b):
            pltpu.make_async_copy(bufs[b], out_hbm.at[0, pl.ds(0, CHUNK)],
                                  ssem[b]).wait()

        for b in range(NBUF):
            start_idx(b, b)
        for b in range(NBUF - 1):
            wait_idx(b)
            start_gather(b)

        def outer_body(g, carry):
            for b in range(NBUF):
                c = g * NBUF + b
                nb = (b + NBUF - 1) % NBUF
                @pl.when(c >= 1)
                def _():
                    wait_scatter(nb)

                @pl.when(c + NBUF - 1 < chunks_per_w)
                def _():
                    wait_idx(nb)
                    start_gather(nb)

                wait_gather(b)

                @pl.when(c + NBUF < chunks_per_w)
                def _():
                    start_idx(b, c + NBUF)

                s = lax.div(gc0 + c, chunks_per_plane)
                pe_regs = [pe_v[s, pl.ds(d * LANES, LANES)]
                           for d in range(n_dreg)]

                def pe_body(r4, carry2):
                    for u in range(4):
                        r = r4 * 4 + u
                        for d in range(n_dreg):
                            sl = pl.ds(d * LANES, LANES)
                            plsc.addupdate(bufs[b].at[r, sl], pe_regs[d])
                    return carry2

                lax.fori_loop(0, CHUNK // 4, pe_body, 0)
                start_scatter(b, c)
            return carry

        lax.fori_loop(0, chunks_per_w // NBUF, outer_body, 0)
        wait_scatter((chunks_per_w - 1) % NBUF)

    return sc_embed


def kernel(x, tok_table):
    batch, seq = x.shape
    assert seq == SEQ
    assert batch % CHUNK == 0
    chunks_per_plane = batch // CHUNK
    total_chunks = seq * chunks_per_plane
    assert total_chunks % NUM_WORKERS == 0
    chunks_per_w = total_chunks // NUM_WORKERS
    x_flat = x.astype(jnp.int32).T.reshape(NUM_WORKERS, chunks_per_w, CHUNK)
    pe = _positional_encoding(SEQ, D_MODEL)
    sc_embed = _make_sc_kernel(batch, chunks_per_plane, chunks_per_w)
    out_sbd = sc_embed(x_flat, tok_table, pe)
    return out_sbd.transpose(1, 0, 2)


# speedup vs baseline: 1.0054x; 1.0054x over previous
import functools

import jax
import jax.numpy as jnp
from jax import lax
from jax.experimental import pallas as pl
from jax.experimental.pallas import tpu as pltpu
from jax.experimental.pallas import tpu_sc as plsc

D_MODEL = 128
SEQ = 50
LANES = 16
NUM_WORKERS = 32
CHUNK = 128
NBUF = 5


def _positional_encoding(seq, d_model):
    pos = jnp.arange(seq, dtype=jnp.float32)[:, None]
    i = jnp.arange(0, d_model, 2, dtype=jnp.float32)
    div = jnp.exp(-i * (jnp.log(10000.0) / d_model))
    ang = pos * div
    pe = jnp.zeros((seq, d_model), dtype=jnp.float32)
    pe = pe.at[:, 0::2].set(jnp.sin(ang))
    pe = pe.at[:, 1::2].set(jnp.cos(ang))
    return pe


def _make_sc_kernel(batch, chunks_per_plane, chunks_per_w):
    mesh = plsc.VectorSubcoreMesh(core_axis_name="c", subcore_axis_name="s")
    n_dreg = D_MODEL // LANES
    assert chunks_per_w % NBUF == 0

    @functools.partial(
        pl.kernel,
        mesh=mesh,
        out_type=jax.ShapeDtypeStruct((SEQ, batch, D_MODEL), jnp.float32),
        scratch_types=[
            pltpu.VMEM((SEQ, D_MODEL), jnp.float32),
        ]
        + [pltpu.VMEM((CHUNK,), jnp.int32) for _ in range(NBUF)]
        + [pltpu.VMEM((CHUNK, D_MODEL), jnp.float32) for _ in range(NBUF)]
        + [pltpu.SemaphoreType.DMA for _ in range(3 * NBUF)],
    )
    def sc_embed(x_hbm, tab_hbm, pe_hbm, out_hbm, pe_v, *bufs_sems):
        ibufs = bufs_sems[:NBUF]
        bufs = bufs_sems[NBUF:2 * NBUF]
        isem = bufs_sems[2 * NBUF:3 * NBUF]
        gsem = bufs_sems[3 * NBUF:4 * NBUF]
        ssem = bufs_sems[4 * NBUF:]
        cid = lax.axis_index("c")
        sid = lax.axis_index("s")
        w = sid * 2 + cid
        pltpu.sync_copy(pe_hbm, pe_v)
        gc0 = w * chunks_per_w

        def start_idx(b, c):
            pltpu.async_copy(x_hbm.at[w, c], ibufs[b], isem[b])

        def wait_idx(b):
            pltpu.make_async_copy(x_hbm.at[w, 0], ibufs[b], isem[b]).wait()

        def start_gather(b):
            pltpu.async_copy(tab_hbm.at[ibufs[b]], bufs[b], gsem[b])

        def wait_gather(b):
            pltpu.make_async_copy(tab_hbm.at[ibufs[b]], bufs[b],
                                  gsem[b]).wait()

        def start_scatter(b, c):
            gc = gc0 + c
            s = lax.div(gc, chunks_per_plane)
            j = lax.rem(gc, chunks_per_plane)
            pltpu.async_copy(bufs[b], out_hbm.at[s, pl.ds(j * CHUNK, CHUNK)],
                             ssem[b])

        def wait_scatter(b):
            pltpu.make_async_copy(bufs[b], out_hbm.at[0, pl.ds(0, CHUNK)],
                                  ssem[b]).wait()

        for b in range(NBUF):
            start_idx(b, b)
        for b in range(NBUF - 1):
            wait_idx(b)
            start_gather(b)

        def outer_body(g, carry):
            for b in range(NBUF):
                c = g * NBUF + b
                nb = (b + NBUF - 1) % NBUF
                @pl.when(c >= 1)
                def _():
                    wait_scatter(nb)

                @pl.when(c + NBUF - 1 < chunks_per_w)
                def _():
                    wait_idx(nb)
                    start_gather(nb)

                wait_gather(b)

                @pl.when(c + NBUF < chunks_per_w)
                def _():
                    start_idx(b, c + NBUF)

                s = lax.div(gc0 + c, chunks_per_plane)
                pe_regs = [pe_v[s, pl.ds(d * LANES, LANES)]
                           for d in range(n_dreg)]

                def pe_body(r2, carry2):
                    for u in range(2):
                        r = r2 * 2 + u
                        for d in range(n_dreg):
                            sl = pl.ds(d * LANES, LANES)
                            plsc.addupdate(bufs[b].at[r, sl], pe_regs[d])
                    return carry2

                lax.fori_loop(0, CHUNK // 2, pe_body, 0)
                start_scatter(b, c)
            return carry

        lax.fori_loop(0, chunks_per_w // NBUF, outer_body, 0)
        wait_scatter((chunks_per_w - 1) % NBUF)

    return sc_embed


def kernel(x, tok_table):
    batch, seq = x.shape
    assert seq == SEQ
    assert batch % CHUNK == 0
    chunks_per_plane = batch // CHUNK
    total_chunks = seq * chunks_per_plane
    assert total_chunks % NUM_WORKERS == 0
    chunks_per_w = total_chunks // NUM_WORKERS
    x_flat = x.astype(jnp.int32).T.reshape(NUM_WORKERS, chunks_per_w, CHUNK)
    pe = _positional_encoding(SEQ, D_MODEL)
    sc_embed = _make_sc_kernel(batch, chunks_per_plane, chunks_per_w)
    out_sbd = sc_embed(x_flat, tok_table, pe)
    return out_sbd.transpose(1, 0, 2)
